# baseline (speedup 1.0000x reference)
import jax
import jax.numpy as jnp
from jax import lax
from jax.experimental import pallas as pl
from jax.experimental.pallas import tpu as pltpu
from jax.experimental.pallas import tpu_sc as plsc

N_NODES = 10000
D = 128
U = 16
H = 8
TW = 144
N_PAD = 10240
NW = 32
K = 76
BLOCKS = 132
BA = BLOCKS + 2
E_PAD = NW * BLOCKS * K
ROWS_PER_TILE = N_PAD // 16
DUMMY = N_NODES
PERIOD = 12
SCAT_BYTES = K * TW * 4
RB = 400


def _stage1_body(x_ref, w_ref, a0_ref, a1_ref, ext_ref, s0_ref):
    xt = jnp.dot(x_ref[...], w_ref[...], preferred_element_type=jnp.float32)
    mask = (lax.broadcasted_iota(jnp.int32, (D, U), 0) // U ==
            lax.broadcasted_iota(jnp.int32, (D, U), 1)).astype(jnp.float32)
    s0 = jnp.dot(xt * a0_ref[...], mask, preferred_element_type=jnp.float32)
    s1 = jnp.dot(xt * a1_ref[...], mask, preferred_element_type=jnp.float32)
    ext_ref[...] = jnp.concatenate([xt, s1], axis=1)
    s0_ref[...] = s0


def _sc_body(ext_hbm, s0_hbm, idx_hbm, out_hbm,
             rows0, rows1, rows2, rows3,
             xtg0, xtg1, xtg2, s0g0, s0g1, s0g2,
             shared,
             semi0, semi1, semi2, semi3,
             semg0, semg1, semg2,
             sems0, sems1, sems2):
    c = lax.axis_index("c")
    s = lax.axis_index("s")
    w = c * 16 + s
    row0 = s * ROWS_PER_TILE
    rows = [rows0, rows1, rows2, rows3]
    xtg = [xtg0, xtg1, xtg2]
    s0g = [s0g0, s0g1, s0g2]
    semi = [semi0, semi1, semi2, semi3]
    semg = [semg0, semg1, semg2]
    sems = [sems0, sems1, sems2]

    zero16 = jnp.zeros((16,), jnp.float32)

    @plsc.parallel_loop(0, K, 1, unroll=4)
    def zrow(r):
        for cc in range(TW // 16):
            xtg0[r, pl.ds(cc * 16, 16)] = zero16

    off = 0
    while off < ROWS_PER_TILE:
        n = min(K, ROWS_PER_TILE - off)
        pltpu.sync_copy(xtg0.at[pl.ds(0, n)],
                        shared.at[pl.ds(row0 + off, n)])
        off += n
    plsc.subcore_barrier()

    def compute_block(xt_b, s0_b):
        @plsc.parallel_loop(0, K, 1, unroll=4)
        def edge(e):
            s1v = xt_b[e, pl.ds(D, 16)]
            s0v = s0_b[e, :]
            t = s0v + s1v
            t = jnp.where(t >= 0.0, t, 0.2 * t)
            t = jnp.minimum(jnp.maximum(t, -2.0), 2.0)
            sc = jnp.exp(t)
            xt_b[e, pl.ds(D, 16)] = sc
            for cc in range(D // 16):
                idx = jnp.full((16,), cc, jnp.int32)
                b = jnp.take_along_axis(sc, idx, axis=0,
                                        mode="promise_in_bounds")
                xt_b[e, pl.ds(cc * 16, 16)] = xt_b[e, pl.ds(cc * 16, 16)] * b

    def wait_xtg(b3, sem_list):
        pltpu.make_async_copy(ext_hbm.at[pl.ds(0, K)], xtg[b3],
                              sem_list[b3]).wait()

    def wait_s0g(b3):
        pltpu.make_async_copy(s0_hbm.at[pl.ds(0, K)], s0g[b3],
                              semg[b3]).wait()

    def wait_rows(b4):
        pltpu.make_async_copy(idx_hbm.at[0, 0, 0], rows[b4].at[0],
                              semi[b4]).wait()
        pltpu.make_async_copy(idx_hbm.at[0, 0, 0], rows[b4].at[1],
                              semi[b4]).wait()

    def start_rows(j, b4):
        pltpu.async_copy(idx_hbm.at[0, w, j], rows[b4].at[0], semi[b4])
        pltpu.async_copy(idx_hbm.at[1, w, j], rows[b4].at[1], semi[b4])

    start_rows(0, 0)
    start_rows(1, 1)
    wait_rows(0)
    pltpu.async_copy(ext_hbm.at[rows[0].at[1]], xtg[0], semg[0])
    pltpu.async_copy(s0_hbm.at[rows[0].at[0]], s0g[0], semg[0])
    pltpu.async_copy(ext_hbm.at[pl.ds(0, K)], xtg[1], sems[1])
    pltpu.async_copy(ext_hbm.at[pl.ds(0, K)], xtg[2], sems[2])

    def outer(i, carry):
        base = i * PERIOD
        for b in range(PERIOD):
            j = base + b
            b3, b4 = b % 3, b % 4
            n3, n4, p4 = (b + 1) % 3, (b + 1) % 4, (b + 2) % 4
            wait_xtg(n3, sems)
            start_rows(j + 2, p4)
            wait_rows(n4)
            pltpu.async_copy(ext_hbm.at[rows[n4].at[1]], xtg[n3], semg[n3])
            pltpu.async_copy(s0_hbm.at[rows[n4].at[0]], s0g[n3], semg[n3])
            wait_xtg(b3, semg)
            wait_s0g(b3)
            compute_block(xtg[b3], s0g[b3])
            pltpu.async_copy(xtg[b3], shared.at[rows[b4].at[0]], sems[b3],
                             add=True)
        return carry

    lax.fori_loop(0, BLOCKS // PERIOD, outer, 0)

    wait_xtg(1, sems)
    wait_xtg(2, sems)
    wait_xtg(0, semg)
    wait_s0g(0)
    wait_rows(1)

    plsc.subcore_barrier()
    pltpu.sync_copy(shared.at[pl.ds(row0, ROWS_PER_TILE)],
                    out_hbm.at[c].at[pl.ds(row0, ROWS_PER_TILE)])


def _stage3_body(p0_ref, p1_ref, ok_ref, o_ref):
    p = p0_ref[0] + p1_ref[0]
    num = p[:, :D]
    sums = p[:, D:D + H]
    inv = jnp.where(sums > 0.0, 1.0 / sums, 0.0)
    emask = (lax.broadcasted_iota(jnp.int32, (H, D), 0) ==
             lax.broadcasted_iota(jnp.int32, (H, D), 1) // U).astype(jnp.float32)
    scale = jnp.dot(inv, emask, preferred_element_type=jnp.float32)
    o = jnp.dot(num * scale, ok_ref[...], preferred_element_type=jnp.float32)
    o_ref[...] = jnp.where(o >= 0.0, o, 0.2 * o)


def kernel(x, edge_index, kernels, att_kernels, out_kernel):
    x32 = x.astype(jnp.float32)
    e2 = edge_index.astype(jnp.int32).T
    e2 = jnp.pad(e2, ((0, 0), (0, E_PAD - e2.shape[1])),
                 constant_values=DUMMY)
    idxr = jnp.pad(e2.reshape(2, NW, BLOCKS, K),
                   ((0, 0), (0, 0), (0, 2), (0, 0)),
                   constant_values=DUMMY)
    w_all = jnp.transpose(kernels, (1, 0, 2)).reshape(D, H * U)
    a0 = att_kernels[:, :U, 0].reshape(1, D)
    a1 = att_kernels[:, U:, 0].reshape(1, D)

    xt_ext, s0_ext = pl.pallas_call(
        _stage1_body,
        grid=(N_NODES // RB,),
        in_specs=[pl.BlockSpec((RB, D), lambda i: (i, 0)),
                  pl.BlockSpec((D, D), lambda i: (0, 0)),
                  pl.BlockSpec((1, D), lambda i: (0, 0)),
                  pl.BlockSpec((1, D), lambda i: (0, 0))],
        out_specs=[pl.BlockSpec((RB, TW), lambda i: (i, 0)),
                   pl.BlockSpec((RB, U), lambda i: (i, 0))],
        out_shape=[jax.ShapeDtypeStruct((N_PAD, TW), jnp.float32),
                   jax.ShapeDtypeStruct((N_PAD, U), jnp.float32)],
    )(x32, w_all, a0, a1)

    partials = pl.kernel(
        _sc_body,
        out_type=jax.ShapeDtypeStruct((2, N_PAD, TW), jnp.float32),
        mesh=plsc.VectorSubcoreMesh(core_axis_name="c", subcore_axis_name="s"),
        compiler_params=pltpu.CompilerParams(use_tc_tiling_on_sc=False),
        scratch_types=(
            [pltpu.VMEM((2, K), jnp.int32)] * 4
            + [pltpu.VMEM((K, TW), jnp.float32)] * 3
            + [pltpu.VMEM((K, U), jnp.float32)] * 3
            + [pltpu.VMEM_SHARED((N_PAD, TW), jnp.float32)]
            + [pltpu.SemaphoreType.DMA] * 10
        ),
    )(xt_ext, s0_ext, idxr)

    return pl.pallas_call(
        _stage3_body,
        grid=(N_NODES // RB,),
        in_specs=[pl.BlockSpec((1, RB, TW), lambda i: (0, i, 0)),
                  pl.BlockSpec((1, RB, TW), lambda i: (1, i, 0)),
                  pl.BlockSpec((D, U), lambda i: (0, 0))],
        out_specs=pl.BlockSpec((RB, U), lambda i: (i, 0)),
        out_shape=jax.ShapeDtypeStruct((N_NODES, U), jnp.float32),
    )(partials, partials, out_kernel)


# speedup vs baseline: 1.2877x; 1.2877x over previous
import jax
import jax.numpy as jnp
from jax import lax
from jax.experimental import pallas as pl
from jax.experimental.pallas import tpu as pltpu
from jax.experimental.pallas import tpu_sc as plsc

N_NODES = 10000
D = 128
U = 16
H = 8
TW = 144
N_PAD = 10240
NW = 32
K = 76
BLOCKS = 132
E_PAD = NW * BLOCKS * K
ROWS_PER_TILE = N_PAD // 16
DUMMY = N_NODES
PERIOD = 12
SCAT_BYTES = K * TW * 4
RB = 400


def _stage1_body(x_ref, w_ref, a0_ref, a1_ref, ext_ref, s0_ref):
    xt = jnp.dot(x_ref[...], w_ref[...], preferred_element_type=jnp.float32)
    mask = (lax.broadcasted_iota(jnp.int32, (D, U), 0) // U ==
            lax.broadcasted_iota(jnp.int32, (D, U), 1)).astype(jnp.float32)
    s0 = jnp.dot(xt * a0_ref[...], mask, preferred_element_type=jnp.float32)
    s1 = jnp.dot(xt * a1_ref[...], mask, preferred_element_type=jnp.float32)
    ext_ref[...] = jnp.concatenate([xt, s1], axis=1)
    s0_ref[...] = s0


def _sc_body(ext_hbm, s0_hbm, idx_hbm, out_hbm,
             rows0, rows1, rows2, rows3,
             xtg0, xtg1, xtg2, s0g0, s0g1, s0g2,
             shared,
             semi0, semi1, semi2, semi3,
             semg0, semg1, semg2,
             sems0, sems1, sems2):
    c = lax.axis_index("c")
    s = lax.axis_index("s")
    w = c * 16 + s
    row0 = s * ROWS_PER_TILE
    rows = [rows0, rows1, rows2, rows3]
    xtg = [xtg0, xtg1, xtg2]
    s0g = [s0g0, s0g1, s0g2]
    semi = [semi0, semi1, semi2, semi3]
    semg = [semg0, semg1, semg2]
    sems = [sems0, sems1, sems2]

    zero16 = jnp.zeros((16,), jnp.float32)

    @plsc.parallel_loop(0, K, 1, unroll=4)
    def zrow(r):
        for cc in range(TW // 16):
            xtg0[r, pl.ds(cc * 16, 16)] = zero16

    off = 0
    while off < ROWS_PER_TILE:
        n = min(K, ROWS_PER_TILE - off)
        pltpu.sync_copy(xtg0.at[pl.ds(0, n)],
                        shared.at[pl.ds(row0 + off, n)])
        off += n
    plsc.subcore_barrier()

    def compute_block(xt_b, s0_b):
        @plsc.parallel_loop(0, K, 1, unroll=4)
        def edge(e):
            s1v = xt_b[e, pl.ds(D, 16)]
            s0v = s0_b[e, :]
            t = s0v + s1v
            t = jnp.where(t >= 0.0, t, 0.2 * t)
            t = jnp.minimum(jnp.maximum(t, -2.0), 2.0)
            sc = jnp.exp(t)
            xt_b[e, pl.ds(D, 16)] = sc
            for cc in range(D // 16):
                idx = jnp.full((16,), cc, jnp.int32)
                b = jnp.take_along_axis(sc, idx, axis=0,
                                        mode="promise_in_bounds")
                xt_b[e, pl.ds(cc * 16, 16)] = xt_b[e, pl.ds(cc * 16, 16)] * b

    def wait_xtg(b3, sem_list):
        pltpu.make_async_copy(ext_hbm.at[pl.ds(0, K)], xtg[b3],
                              sem_list[b3]).wait()

    def wait_s0g(b3):
        pltpu.make_async_copy(s0_hbm.at[pl.ds(0, K)], s0g[b3],
                              semg[b3]).wait()

    def wait_rows(b4):
        pltpu.make_async_copy(idx_hbm.at[0, 0, 0], rows[b4].at[0],
                              semi[b4]).wait()
        pltpu.make_async_copy(idx_hbm.at[0, 0, 0], rows[b4].at[1],
                              semi[b4]).wait()

    def start_rows(j, b4):
        pltpu.async_copy(idx_hbm.at[0, w, j], rows[b4].at[0], semi[b4])
        pltpu.async_copy(idx_hbm.at[1, w, j], rows[b4].at[1], semi[b4])

    start_rows(0, 0)
    start_rows(1, 1)
    wait_rows(0)
    pltpu.async_copy(ext_hbm.at[rows[0].at[1]], xtg[0], semg[0])
    pltpu.async_copy(s0_hbm.at[rows[0].at[0]], s0g[0], semg[0])
    pltpu.async_copy(ext_hbm.at[pl.ds(0, K)], xtg[1], sems[1])
    pltpu.async_copy(ext_hbm.at[pl.ds(0, K)], xtg[2], sems[2])

    def sub_body(j, b, do_rows, do_gather):
        b3, b4 = b % 3, b % 4
        n3, n4, p4 = (b + 1) % 3, (b + 1) % 4, (b + 2) % 4
        if do_gather:
            wait_xtg(n3, sems)
        if do_rows:
            start_rows(j + 2, p4)
        if do_gather:
            wait_rows(n4)
            pltpu.async_copy(ext_hbm.at[rows[n4].at[1]], xtg[n3], semg[n3])
            pltpu.async_copy(s0_hbm.at[rows[n4].at[0]], s0g[n3], semg[n3])
        wait_xtg(b3, semg)
        wait_s0g(b3)
        compute_block(xtg[b3], s0g[b3])
        pltpu.async_copy(xtg[b3], shared.at[rows[b4].at[0]], sems[b3],
                         add=True)

    def outer(i, carry):
        base = i * PERIOD
        for b in range(PERIOD):
            sub_body(base + b, b, True, True)
        return carry

    lax.fori_loop(0, BLOCKS // PERIOD - 1, outer, 0)
    for b in range(PERIOD):
        j = BLOCKS - PERIOD + b
        sub_body(j, b, j + 2 < BLOCKS, j + 1 < BLOCKS)

    wait_xtg(0, sems)
    wait_xtg(1, sems)
    wait_xtg(2, sems)

    plsc.subcore_barrier()
    pltpu.sync_copy(shared.at[pl.ds(row0, ROWS_PER_TILE)],
                    out_hbm.at[c].at[pl.ds(row0, ROWS_PER_TILE)])


def _stage3_body(p0_ref, p1_ref, ok_ref, o_ref):
    p = p0_ref[0] + p1_ref[0]
    num = p[:, :D]
    sums = p[:, D:D + H]
    inv = jnp.where(sums > 0.0, 1.0 / sums, 0.0)
    emask = (lax.broadcasted_iota(jnp.int32, (H, D), 0) ==
             lax.broadcasted_iota(jnp.int32, (H, D), 1) // U).astype(jnp.float32)
    scale = jnp.dot(inv, emask, preferred_element_type=jnp.float32)
    o = jnp.dot(num * scale, ok_ref[...], preferred_element_type=jnp.float32)
    o_ref[...] = jnp.where(o >= 0.0, o, 0.2 * o)


def kernel(x, edge_index, kernels, att_kernels, out_kernel):
    x32 = x.astype(jnp.float32)
    e2 = edge_index.astype(jnp.int32).T
    e2 = jnp.pad(e2, ((0, 0), (0, E_PAD - e2.shape[1])),
                 constant_values=DUMMY)
    idxr = e2.reshape(2, NW, BLOCKS, K)
    w_all = jnp.transpose(kernels, (1, 0, 2)).reshape(D, H * U)
    a0 = att_kernels[:, :U, 0].reshape(1, D)
    a1 = att_kernels[:, U:, 0].reshape(1, D)

    xt_ext, s0_ext = pl.pallas_call(
        _stage1_body,
        grid=(N_NODES // RB,),
        in_specs=[pl.BlockSpec((RB, D), lambda i: (i, 0)),
                  pl.BlockSpec((D, D), lambda i: (0, 0)),
                  pl.BlockSpec((1, D), lambda i: (0, 0)),
                  pl.BlockSpec((1, D), lambda i: (0, 0))],
        out_specs=[pl.BlockSpec((RB, TW), lambda i: (i, 0)),
                   pl.BlockSpec((RB, U), lambda i: (i, 0))],
        out_shape=[jax.ShapeDtypeStruct((N_PAD, TW), jnp.float32),
                   jax.ShapeDtypeStruct((N_PAD, U), jnp.float32)],
    )(x32, w_all, a0, a1)

    partials = pl.kernel(
        _sc_body,
        out_type=jax.ShapeDtypeStruct((2, N_PAD, TW), jnp.float32),
        mesh=plsc.VectorSubcoreMesh(core_axis_name="c", subcore_axis_name="s"),
        compiler_params=pltpu.CompilerParams(use_tc_tiling_on_sc=False),
        scratch_types=(
            [pltpu.VMEM((2, K), jnp.int32)] * 4
            + [pltpu.VMEM((K, TW), jnp.float32)] * 3
            + [pltpu.VMEM((K, U), jnp.float32)] * 3
            + [pltpu.VMEM_SHARED((N_PAD, TW), jnp.float32)]
            + [pltpu.SemaphoreType.DMA] * 10
        ),
    )(xt_ext, s0_ext, idxr)

    return pl.pallas_call(
        _stage3_body,
        grid=(N_NODES // RB,),
        in_specs=[pl.BlockSpec((1, RB, TW), lambda i: (0, i, 0)),
                  pl.BlockSpec((1, RB, TW), lambda i: (1, i, 0)),
                  pl.BlockSpec((D, U), lambda i: (0, 0))],
        out_specs=pl.BlockSpec((RB, U), lambda i: (i, 0)),
        out_shape=jax.ShapeDtypeStruct((N_NODES, U), jnp.float32),
    )(partials, partials, out_kernel)
